# baseline (speedup 1.0000x reference)
import functools

import jax
import jax.numpy as jnp
from jax import lax
from jax.experimental import pallas as pl
from jax.experimental.pallas import tpu as pltpu
from jax.experimental.pallas import tpu_sc as plsc

_VOCAB = 1000000
_DIM = 64
_BATCH = 4096
_HIST = 50

_B = _BATCH * _HIST
_NC = 2
_NS = 16
_NW = _NC * _NS
_BPW = _B // _NW
_CHUNK = 128
_NCH = _BPW // _CHUNK

_mesh = plsc.VectorSubcoreMesh(core_axis_name="c", subcore_axis_name="s")


@functools.partial(
    pl.kernel,
    out_type=jax.ShapeDtypeStruct((_B, _DIM), jnp.float32),
    mesh=_mesh,
    scratch_types=[
        pltpu.VMEM((_NCH, _CHUNK), jnp.int32),
        pltpu.VMEM((_CHUNK, _DIM), jnp.float32),
        pltpu.SemaphoreType.DMA,
    ],
    compiler_params=pltpu.CompilerParams(use_tc_tiling_on_sc=False),
)
def _sc_gather(idx_hbm, table_hbm, out_hbm, idx_v, rows_v, sem):
    wid = lax.axis_index("s") * _NC + lax.axis_index("c")
    base = wid * _BPW
    pltpu.sync_copy(idx_hbm.at[wid], idx_v)

    def step(j, carry):
        pltpu.async_copy(table_hbm.at[idx_v.at[j]], rows_v, sem).wait()
        pltpu.sync_copy(rows_v, out_hbm.at[pl.ds(base + j * _CHUNK, _CHUNK)])
        return carry

    lax.fori_loop(0, _NCH, step, 0)


def kernel(inputs, embeddings):
    idx = inputs.astype(jnp.int32).reshape(_NW, _NCH, _CHUNK)
    out = _sc_gather(idx, embeddings)
    return out.reshape(_BATCH, _HIST, _DIM)


# speedup vs baseline: 1.0425x; 1.0425x over previous
import functools

import jax
import jax.numpy as jnp
from jax import lax
from jax.experimental import pallas as pl
from jax.experimental.pallas import tpu as pltpu
from jax.experimental.pallas import tpu_sc as plsc

_VOCAB = 1000000
_DIM = 64
_BATCH = 4096
_HIST = 50

_B = _BATCH * _HIST
_NC = 2
_NS = 16
_NW = _NC * _NS
_BPW = _B // _NW
_CHUNK = 128
_NCH = _BPW // _CHUNK
_K = 5
_GROUP = _K * _CHUNK
_NG = _NCH // _K

_mesh = plsc.VectorSubcoreMesh(core_axis_name="c", subcore_axis_name="s")


@functools.partial(
    pl.kernel,
    out_type=jax.ShapeDtypeStruct((_B, _DIM), jnp.float32),
    mesh=_mesh,
    scratch_types=[
        pltpu.VMEM((_NCH, _CHUNK), jnp.int32),
        pltpu.VMEM((_GROUP, _DIM), jnp.float32),
        pltpu.VMEM((_GROUP, _DIM), jnp.float32),
        pltpu.SemaphoreType.DMA,
        pltpu.SemaphoreType.DMA,
        pltpu.SemaphoreType.DMA,
        pltpu.SemaphoreType.DMA,
    ],
    compiler_params=pltpu.CompilerParams(use_tc_tiling_on_sc=False),
)
def _sc_gather(idx_hbm, table_hbm, out_hbm, idx_v, rows_a, rows_b,
               gsem_a, gsem_b, osem_a, osem_b):
    wid = lax.axis_index("s") * _NC + lax.axis_index("c")
    base = wid * _BPW
    pltpu.sync_copy(idx_hbm.at[wid], idx_v)

    def fire_gathers(g, rows, sem):
        for b in range(_K):
            pltpu.async_copy(table_hbm.at[idx_v.at[g * _K + b]],
                             rows.at[pl.ds(b * _CHUNK, _CHUNK)], sem)

    def drain(rows, sem):
        pltpu.make_async_copy(out_hbm.at[pl.ds(0, _GROUP)], rows, sem).wait()

    def fire_out(g, rows, sem):
        pltpu.async_copy(rows, out_hbm.at[pl.ds(base + g * _GROUP, _GROUP)],
                         sem)

    def drain_out(rows, sem):
        pltpu.make_async_copy(rows, out_hbm.at[pl.ds(0, _GROUP)], sem).wait()

    fire_gathers(0, rows_a, gsem_a)
    fire_gathers(1, rows_b, gsem_b)
    drain(rows_a, gsem_a)
    fire_out(0, rows_a, osem_a)

    def body(i, carry):
        g = 1 + 2 * i
        drain(rows_b, gsem_b)
        fire_out(g, rows_b, osem_b)
        drain_out(rows_a, osem_a)
        fire_gathers(g + 1, rows_a, gsem_a)
        drain(rows_a, gsem_a)
        fire_out(g + 1, rows_a, osem_a)
        drain_out(rows_b, osem_b)
        fire_gathers(g + 2, rows_b, gsem_b)
        return carry

    lax.fori_loop(0, _NG // 2 - 1, body, 0)

    drain(rows_b, gsem_b)
    fire_out(_NG - 1, rows_b, osem_b)
    drain_out(rows_a, osem_a)
    drain_out(rows_b, osem_b)


def kernel(inputs, embeddings):
    idx = inputs.astype(jnp.int32).reshape(_NW, _NCH, _CHUNK)
    out = _sc_gather(idx, embeddings)
    return out.reshape(_BATCH, _HIST, _DIM)
